# baseline (speedup 1.0000x reference)
import jax
import jax.numpy as jnp
from jax.experimental import pallas as pl
from jax.experimental.pallas import tpu as pltpu

D_MODEL = 1024
SEQ = 8192
BATCH = 2

_SPINE = [0, 2, 4, 12, 36, 104, 304, 888, 2592, 7568]
_WRITE_POS = _SPINE[3:]
_NW = len(_WRITE_POS)

_CHUNK = 1024
_NCHUNK = (BATCH * SEQ) // _CHUNK


def _fused_kernel(x_ref, rows_ref, axz_ref, bxy_ref, gx_ref, awy_ref,
                  bwx_ref, gw_ref, axv_ref, bwv_ref, gv_ref, gwv_ref,
                  gwz_ref, gb_ref, lnw_ref, lnb_ref, out_ref,
                  new_rows, copy_sem, scat_sem):
    for c in range(_NCHUNK):
        pltpu.make_async_copy(
            x_ref.at[pl.ds(c * _CHUNK, _CHUNK), :],
            out_ref.at[pl.ds(c * _CHUNK, _CHUNK), :],
            copy_sem,
        ).start()

    w_v = gwv_ref[...]
    w_z = gwz_ref[...]
    gb = gb_ref[...]
    lnw = lnw_ref[...]
    lnb = lnb_ref[...]
    vals = [rows_ref[:, i, :] for i in range(len(_SPINE))]
    for k in range(3, len(_SPINE)):
        z = vals[k]
        y = vals[k - 1]
        x_prev = vals[k - 2]
        x_new = axz_ref[...] * z + bxy_ref[...] * y + gx_ref[...]
        w = awy_ref[...] * y + bwx_ref[...] * x_prev + gw_ref[...]
        v = axv_ref[...] * x_new + bwv_ref[...] * w + gv_ref[...]
        logits = (jnp.dot(v, w_v, preferred_element_type=jnp.float32)
                  + jnp.dot(z, w_z, preferred_element_type=jnp.float32) + gb)
        g = jax.nn.sigmoid(logits)
        gated = g * v + (1.0 - g) * z
        mean = jnp.mean(gated, axis=-1, keepdims=True)
        var = jnp.mean((gated - mean) ** 2, axis=-1, keepdims=True)
        vals[k] = (gated - mean) * jax.lax.rsqrt(var + 1e-5) * lnw + lnb
    for j, k in enumerate(range(3, len(_SPINE))):
        for b in range(BATCH):
            new_rows[b * _NW + j, :] = vals[k][b, :]

    for c in range(_NCHUNK):
        pltpu.make_async_copy(
            x_ref.at[pl.ds(c * _CHUNK, _CHUNK), :],
            out_ref.at[pl.ds(c * _CHUNK, _CHUNK), :],
            copy_sem,
        ).wait()
    for b in range(BATCH):
        for j, p in enumerate(_WRITE_POS):
            pltpu.make_async_copy(
                new_rows.at[pl.ds(b * _NW + j, 1), :],
                out_ref.at[pl.ds(b * SEQ + p, 1), :],
                scat_sem,
            ).start()
    for b in range(BATCH):
        for j, p in enumerate(_WRITE_POS):
            pltpu.make_async_copy(
                new_rows.at[pl.ds(b * _NW + j, 1), :],
                out_ref.at[pl.ds(b * SEQ + p, 1), :],
                scat_sem,
            ).wait()


def kernel(x, alpha_xz, beta_xy, gamma_x, alpha_wy, beta_wx, gamma_w,
           alpha_xv, beta_wv, gamma_v, gate_w, gate_b, ln_w, ln_b):
    spine_rows = x[:, jnp.array(_SPINE), :]
    w_v = gate_w[:, :D_MODEL].T
    w_z = gate_w[:, D_MODEL:].T
    x_flat = x.reshape(BATCH * SEQ, D_MODEL)

    vspec = pl.BlockSpec(memory_space=pltpu.MemorySpace.VMEM)
    out_flat = pl.pallas_call(
        _fused_kernel,
        in_specs=[pl.BlockSpec(memory_space=pltpu.MemorySpace.HBM)]
        + [vspec] * 15,
        out_specs=pl.BlockSpec(memory_space=pltpu.MemorySpace.HBM),
        out_shape=jax.ShapeDtypeStruct((BATCH * SEQ, D_MODEL), jnp.float32),
        scratch_shapes=[
            pltpu.VMEM((BATCH * _NW, D_MODEL), jnp.float32),
            pltpu.SemaphoreType.DMA,
            pltpu.SemaphoreType.DMA,
        ],
    )(x_flat, spine_rows, alpha_xz, beta_xy, gamma_x, alpha_wy, beta_wx,
      gamma_w, alpha_xv, beta_wv, gamma_v, w_v, w_z, gate_b, ln_w, ln_b)
    return out_flat.reshape(BATCH, SEQ, D_MODEL)


# speedup vs baseline: 29.9278x; 29.9278x over previous
import jax
import jax.numpy as jnp
from jax.experimental import pallas as pl
from jax.experimental.pallas import tpu as pltpu

D_MODEL = 1024
SEQ = 8192
BATCH = 2

_SPINE = [0, 2, 4, 12, 36, 104, 304, 888, 2592, 7568]
_WRITE_POS = _SPINE[3:]
_NW = len(_WRITE_POS)

_BLK = 512
_NBLK = SEQ // _BLK
_PER_BLOCK = {}
for _j, _p in enumerate(_WRITE_POS):
    _PER_BLOCK.setdefault(_p // _BLK, []).append((_p % _BLK, _j))


def _fused_kernel(x_ref, rows_ref, axz_ref, bxy_ref, gx_ref, awy_ref,
                  bwx_ref, gw_ref, axv_ref, bwv_ref, gv_ref, gwv_ref,
                  gwz_ref, gb_ref, lnw_ref, lnb_ref, out_ref, new_rows):
    pid = pl.program_id(0)

    @pl.when(pid == 0)
    def _recurrence():
        w_v = gwv_ref[...]
        w_z = gwz_ref[...]
        gb = gb_ref[...]
        lnw = lnw_ref[...]
        lnb = lnb_ref[...]
        vals = [rows_ref[:, i, :] for i in range(len(_SPINE))]
        for k in range(3, len(_SPINE)):
            z = vals[k]
            y = vals[k - 1]
            x_prev = vals[k - 2]
            x_new = axz_ref[...] * z + bxy_ref[...] * y + gx_ref[...]
            w = awy_ref[...] * y + bwx_ref[...] * x_prev + gw_ref[...]
            v = axv_ref[...] * x_new + bwv_ref[...] * w + gv_ref[...]
            logits = (jnp.dot(v, w_v, preferred_element_type=jnp.float32)
                      + jnp.dot(z, w_z, preferred_element_type=jnp.float32)
                      + gb)
            g = jax.nn.sigmoid(logits)
            gated = g * v + (1.0 - g) * z
            mean = jnp.mean(gated, axis=-1, keepdims=True)
            var = jnp.mean((gated - mean) ** 2, axis=-1, keepdims=True)
            vals[k] = (gated - mean) * jax.lax.rsqrt(var + 1e-5) * lnw + lnb
        for j, k in enumerate(range(3, len(_SPINE))):
            new_rows[:, j, :] = vals[k]

    out_ref[...] = x_ref[...]
    for b, lst in _PER_BLOCK.items():
        @pl.when(pid == b)
        def _scatter(lst=lst):
            for off, j in lst:
                out_ref[:, off, :] = new_rows[:, j, :]


def kernel(x, alpha_xz, beta_xy, gamma_x, alpha_wy, beta_wx, gamma_w,
           alpha_xv, beta_wv, gamma_v, gate_w, gate_b, ln_w, ln_b):
    spine_rows = x[:, jnp.array(_SPINE), :]
    w_v = gate_w[:, :D_MODEL].T
    w_z = gate_w[:, D_MODEL:].T

    cspec = pl.BlockSpec((BATCH, _BLK, D_MODEL), lambda i: (0, i, 0))
    full = lambda shape: pl.BlockSpec(shape, lambda i: (0,) * len(shape))
    vec = full((D_MODEL,))
    out = pl.pallas_call(
        _fused_kernel,
        grid=(_NBLK,),
        in_specs=[
            cspec,
            full((BATCH, len(_SPINE), D_MODEL)),
            vec, vec, vec, vec, vec, vec, vec, vec, vec,
            full((D_MODEL, D_MODEL)), full((D_MODEL, D_MODEL)),
            vec, vec, vec,
        ],
        out_specs=cspec,
        out_shape=jax.ShapeDtypeStruct((BATCH, SEQ, D_MODEL), jnp.float32),
        scratch_shapes=[pltpu.VMEM((BATCH, _NW, D_MODEL), jnp.float32)],
        compiler_params=pltpu.CompilerParams(
            dimension_semantics=("arbitrary",)),
    )(x, spine_rows, alpha_xz, beta_xy, gamma_x, alpha_wy, beta_wx, gamma_w,
      alpha_xv, beta_wv, gamma_v, w_v, w_z, gate_b, ln_w, ln_b)
    return out


# speedup vs baseline: 30.8460x; 1.0307x over previous
import jax
import jax.numpy as jnp
from jax.experimental import pallas as pl
from jax.experimental.pallas import tpu as pltpu

D_MODEL = 1024
SEQ = 8192
BATCH = 2

_SPINE = [0, 2, 4, 12, 36, 104, 304, 888, 2592, 7568]
_WRITE_POS = _SPINE[3:]
_NW = len(_WRITE_POS)

_BLK = 2048
_NBLK = (BATCH * SEQ) // _BLK
_PER_BLOCK = {}
for _b in range(BATCH):
    for _j, _p in enumerate(_WRITE_POS):
        _f = _b * SEQ + _p
        _PER_BLOCK.setdefault(_f // _BLK, []).append((_f % _BLK, _b, _j))


def _fused_kernel(x_ref, rows_ref, axz_ref, bxy_ref, gx_ref, awy_ref,
                  bwx_ref, gw_ref, axv_ref, bwv_ref, gv_ref, gwv_ref,
                  gwz_ref, gb_ref, lnw_ref, lnb_ref, out_ref, new_rows):
    pid = pl.program_id(0)

    @pl.when(pid == 0)
    def _recurrence():
        w_v = gwv_ref[...]
        w_z = gwz_ref[...]
        gb = gb_ref[...]
        lnw = lnw_ref[...]
        lnb = lnb_ref[...]
        vals = [rows_ref[:, i, :] for i in range(len(_SPINE))]
        for k in range(3, len(_SPINE)):
            z = vals[k]
            y = vals[k - 1]
            x_prev = vals[k - 2]
            x_new = axz_ref[...] * z + bxy_ref[...] * y + gx_ref[...]
            w = awy_ref[...] * y + bwx_ref[...] * x_prev + gw_ref[...]
            v = axv_ref[...] * x_new + bwv_ref[...] * w + gv_ref[...]
            logits = (jnp.dot(v, w_v, preferred_element_type=jnp.float32)
                      + jnp.dot(z, w_z, preferred_element_type=jnp.float32)
                      + gb)
            g = jax.nn.sigmoid(logits)
            gated = g * v + (1.0 - g) * z
            mean = jnp.mean(gated, axis=-1, keepdims=True)
            var = jnp.mean((gated - mean) ** 2, axis=-1, keepdims=True)
            vals[k] = (gated - mean) * jax.lax.rsqrt(var + 1e-5) * lnw + lnb
        for j, k in enumerate(range(3, len(_SPINE))):
            new_rows[:, j, :] = vals[k]

    out_ref[...] = x_ref[...]
    for b, lst in _PER_BLOCK.items():
        @pl.when(pid == b)
        def _scatter(lst=lst):
            for off, bb, j in lst:
                out_ref[off, :] = new_rows[bb, j, :]


def kernel(x, alpha_xz, beta_xy, gamma_x, alpha_wy, beta_wx, gamma_w,
           alpha_xv, beta_wv, gamma_v, gate_w, gate_b, ln_w, ln_b):
    spine_rows = x[:, jnp.array(_SPINE), :]
    w_v = gate_w[:, :D_MODEL].T
    w_z = gate_w[:, D_MODEL:].T
    x_flat = x.reshape(BATCH * SEQ, D_MODEL)

    cspec = pl.BlockSpec((_BLK, D_MODEL), lambda i: (i, 0))
    full = lambda shape: pl.BlockSpec(shape, lambda i: (0,) * len(shape))
    vec = full((D_MODEL,))
    out = pl.pallas_call(
        _fused_kernel,
        grid=(_NBLK,),
        in_specs=[
            cspec,
            full((BATCH, len(_SPINE), D_MODEL)),
            vec, vec, vec, vec, vec, vec, vec, vec, vec,
            full((D_MODEL, D_MODEL)), full((D_MODEL, D_MODEL)),
            vec, vec, vec,
        ],
        out_specs=cspec,
        out_shape=jax.ShapeDtypeStruct((BATCH * SEQ, D_MODEL), jnp.float32),
        scratch_shapes=[pltpu.VMEM((BATCH, _NW, D_MODEL), jnp.float32)],
        compiler_params=pltpu.CompilerParams(
            dimension_semantics=("arbitrary",)),
    )(x_flat, spine_rows, alpha_xz, beta_xy, gamma_x, alpha_wy, beta_wx,
      gamma_w, alpha_xv, beta_wv, gamma_v, w_v, w_z, gate_b, ln_w, ln_b)
    return out.reshape(BATCH, SEQ, D_MODEL)


# speedup vs baseline: 32.2138x; 1.0443x over previous
import jax
import jax.numpy as jnp
from jax.experimental import pallas as pl
from jax.experimental.pallas import tpu as pltpu

D_MODEL = 1024
SEQ = 8192
BATCH = 2

_SPINE = [0, 2, 4, 12, 36, 104, 304, 888, 2592, 7568]
_WRITE_POS = _SPINE[3:]
_NW = len(_WRITE_POS)

_CH = 2048
_NC = (BATCH * SEQ) // _CH
_NS = 4


def _fused_kernel(x_ref, rows_ref, axz_ref, bxy_ref, gx_ref, awy_ref,
                  bwx_ref, gw_ref, axv_ref, bwv_ref, gv_ref, gwv_ref,
                  gwz_ref, gb_ref, lnw_ref, lnb_ref, out_ref,
                  buf, new_rows, in_sems, out_sems, scat_sem):
    def in_copy(c):
        return pltpu.make_async_copy(
            x_ref.at[pl.ds(c * _CH, _CH), :], buf.at[c % _NS],
            in_sems.at[c % _NS])

    def out_copy(c):
        return pltpu.make_async_copy(
            buf.at[c % _NS], out_ref.at[pl.ds(c * _CH, _CH), :],
            out_sems.at[c % _NS])

    for c in range(min(_NS, _NC)):
        in_copy(c).start()

    w_v = gwv_ref[...]
    w_z = gwz_ref[...]
    gb = gb_ref[...]
    lnw = lnw_ref[...]
    lnb = lnb_ref[...]
    vals = [rows_ref[:, i, :] for i in range(len(_SPINE))]
    for k in range(3, len(_SPINE)):
        z = vals[k]
        y = vals[k - 1]
        x_prev = vals[k - 2]
        x_new = axz_ref[...] * z + bxy_ref[...] * y + gx_ref[...]
        w = awy_ref[...] * y + bwx_ref[...] * x_prev + gw_ref[...]
        v = axv_ref[...] * x_new + bwv_ref[...] * w + gv_ref[...]
        logits = (jnp.dot(v, w_v, preferred_element_type=jnp.float32)
                  + jnp.dot(z, w_z, preferred_element_type=jnp.float32)
                  + gb)
        g = jax.nn.sigmoid(logits)
        gated = g * v + (1.0 - g) * z
        mean = jnp.mean(gated, axis=-1, keepdims=True)
        var = jnp.mean((gated - mean) ** 2, axis=-1, keepdims=True)
        vals[k] = (gated - mean) * jax.lax.rsqrt(var + 1e-5) * lnw + lnb
    for j, k in enumerate(range(3, len(_SPINE))):
        for b in range(BATCH):
            new_rows[b * _NW + j, :] = vals[k][b, :]

    for c in range(_NC):
        in_copy(c).wait()
        out_copy(c).start()
        nxt = c + _NS
        if nxt < _NC:
            out_copy(c).wait()
            in_copy(nxt).start()
    for c in range(max(0, _NC - _NS), _NC):
        out_copy(c).wait()

    for b in range(BATCH):
        for j, p in enumerate(_WRITE_POS):
            pltpu.make_async_copy(
                new_rows.at[pl.ds(b * _NW + j, 1), :],
                out_ref.at[pl.ds(b * SEQ + p, 1), :],
                scat_sem,
            ).start()
    for b in range(BATCH):
        for j, p in enumerate(_WRITE_POS):
            pltpu.make_async_copy(
                new_rows.at[pl.ds(b * _NW + j, 1), :],
                out_ref.at[pl.ds(b * SEQ + p, 1), :],
                scat_sem,
            ).wait()


def kernel(x, alpha_xz, beta_xy, gamma_x, alpha_wy, beta_wx, gamma_w,
           alpha_xv, beta_wv, gamma_v, gate_w, gate_b, ln_w, ln_b):
    spine_rows = x[:, jnp.array(_SPINE), :]
    w_v = gate_w[:, :D_MODEL].T
    w_z = gate_w[:, D_MODEL:].T
    x_flat = x.reshape(BATCH * SEQ, D_MODEL)

    vspec = pl.BlockSpec(memory_space=pltpu.MemorySpace.VMEM)
    out_flat = pl.pallas_call(
        _fused_kernel,
        in_specs=[pl.BlockSpec(memory_space=pltpu.MemorySpace.HBM)]
        + [vspec] * 15,
        out_specs=pl.BlockSpec(memory_space=pltpu.MemorySpace.HBM),
        out_shape=jax.ShapeDtypeStruct((BATCH * SEQ, D_MODEL), jnp.float32),
        scratch_shapes=[
            pltpu.VMEM((_NS, _CH, D_MODEL), jnp.float32),
            pltpu.VMEM((BATCH * _NW, D_MODEL), jnp.float32),
            pltpu.SemaphoreType.DMA((_NS,)),
            pltpu.SemaphoreType.DMA((_NS,)),
            pltpu.SemaphoreType.DMA,
        ],
    )(x_flat, spine_rows, alpha_xz, beta_xy, gamma_x, alpha_wy, beta_wx,
      gamma_w, alpha_xv, beta_wv, gamma_v, w_v, w_z, gate_b, ln_w, ln_b)
    return out_flat.reshape(BATCH, SEQ, D_MODEL)
